# Initial kernel scaffold; baseline (speedup 1.0000x reference)
import jax
import jax.numpy as jnp
from jax.experimental import pallas as pl


def kernel(X, lengths, W_l, b_l, w_d, b_d, W_o, b_o):
    raise NotImplementedError("write your pallas kernel here")


# speedup vs baseline: 13.7056x; 13.7056x over previous
import jax
import jax.numpy as jnp
from jax.experimental import pallas as pl
from jax.experimental.pallas import tpu as pltpu

B, T, F, H, C = 4, 256, 64, 128, 2


def _body(t_s, m_s, x_s, d_s, len_s, wd_s, bd_s,
          W0, W1, bl, WoT, bo_s, out_ref, h_ref):
    h_ref[...] = jnp.zeros((B, F, H), dtype=jnp.float32)

    def step(j, carry):
        c_all, s_all, cnt_all = carry
        c_rows, s_rows, cnt_rows = [], [], []
        for b in range(B):
            c_b = c_all[b:b + 1]
            s_b = s_all[b:b + 1]
            cnt_b = cnt_all[b:b + 1]
            L_b = len_s[b]
            valid = j < L_b
            tj = t_s[b, j]
            tp = t_s[b, jnp.maximum(j - 1, 0)]
            ng = valid & (j > 0) & (tj != tp)
            c_b = jnp.where(ng, s_b / cnt_b, c_b)
            s_b = jnp.where(ng, jnp.zeros_like(s_b), s_b)
            cnt_b = jnp.where(ng, jnp.zeros_like(cnt_b), cnt_b)

            fi = m_s[b, j]
            xj = x_s[b, j]
            dv = wd_s[fi] * d_s[b, j] + bd_s[fi]
            decay = jnp.exp(-jnp.maximum(jnp.full((1, H), dv), 0.0))
            h_f = h_ref[b, pl.ds(fi, 1), :]
            h_dec = decay * h_f
            w1 = W1[fi]
            out = jax.lax.dot_general(
                h_dec, w1, (((1,), (0,)), ((), ())),
                preferred_element_type=jnp.float32)
            out = out + xj * W0[pl.ds(fi, 1), :] + bl[pl.ds(fi, 1), :]
            gi = jax.nn.sigmoid(out[:, 0:H])
            gf = jax.nn.sigmoid(out[:, H:2 * H])
            go = jax.nn.sigmoid(out[:, 2 * H:3 * H])
            gc = jnp.tanh(out[:, 3 * H:4 * H])
            cell = gf * c_b + gi * gc
            hnew = go * jnp.tanh(cell)
            h_ref[b, pl.ds(fi, 1), :] = jnp.where(valid, hnew, h_f)
            s_b = s_b + jnp.where(valid, cell, jnp.zeros_like(cell))
            cnt_b = cnt_b + jnp.where(valid, jnp.full((1, H), 1.0),
                                      jnp.zeros((1, H), jnp.float32))
            c_rows.append(c_b)
            s_rows.append(s_b)
            cnt_rows.append(cnt_b)
        return (jnp.concatenate(c_rows, axis=0),
                jnp.concatenate(s_rows, axis=0),
                jnp.concatenate(cnt_rows, axis=0))

    init = (jnp.zeros((B, H), jnp.float32),
            jnp.zeros((B, H), jnp.float32),
            jnp.zeros((B, H), jnp.float32))
    c_all, s_all, cnt_all = jax.lax.fori_loop(0, T, step, init)
    c_all = s_all / cnt_all

    for b in range(B):
        feat = jnp.concatenate([c_all[b:b + 1], h_ref[b]], axis=0)
        l0 = jnp.sum(feat * WoT[0])
        l1 = jnp.sum(feat * WoT[1])
        d = (l1 - l0) + (bo_s[1] - bo_s[0])
        p1 = jax.nn.sigmoid(jnp.full((1, H), d))
        out_ref[b:b + 1, 0:1] = (1.0 - p1)[:, 0:1]
        out_ref[b:b + 1, 1:2] = p1[:, 0:1]


def kernel(X, lengths, W_l, b_l, w_d, b_d, W_o, b_o):
    t = X[:, 0, :]
    m = X[:, 1, :].astype(jnp.int32)
    x = X[:, 2, :]
    delt = X[:, 3, :]
    W0 = W_l[:, 0, :]
    W1 = W_l[:, 1:, :]
    WoT = W_o.reshape(F + 1, H, C).transpose(2, 0, 1)

    smem = pl.BlockSpec(memory_space=pltpu.SMEM)
    vmem = pl.BlockSpec(memory_space=pltpu.VMEM)
    out = pl.pallas_call(
        _body,
        out_shape=jax.ShapeDtypeStruct((B, C), jnp.float32),
        in_specs=[smem, smem, smem, smem, smem, smem, smem,
                  vmem, vmem, vmem, vmem, smem],
        out_specs=pl.BlockSpec(memory_space=pltpu.VMEM),
        scratch_shapes=[pltpu.VMEM((B, F, H), jnp.float32)],
    )(t, m, x, delt, lengths.astype(jnp.int32), w_d, b_d,
      W0, W1, b_l, WoT, b_o)
    return out


# speedup vs baseline: 31.7273x; 2.3149x over previous
import jax
import jax.numpy as jnp
from jax.experimental import pallas as pl
from jax.experimental.pallas import tpu as pltpu

B, T, F, H, C = 4, 256, 64, 128, 2


def _rows(scalars, width=H):
    return jnp.concatenate(
        [jnp.full((1, width), s, jnp.float32) for s in scalars], axis=0)


def _body(t_s, m_s, x_s, d_s, len_s, wd_s, bd_s,
          W0, W1, bl, WoT, bo_s, out_ref, h_ref):
    h_ref[...] = jnp.zeros((B, F, H), dtype=jnp.float32)
    n_steps = jnp.maximum(jnp.maximum(len_s[0], len_s[1]),
                          jnp.maximum(len_s[2], len_s[3]))

    def step(j, carry):
        c_all, s_all, cnt_all = carry
        valid_l, ng_l, fi_l, dv_l, xj_l = [], [], [], [], []
        for b in range(B):
            valid = j < len_s[b]
            tj = t_s[b, j]
            tp = t_s[b, jnp.maximum(j - 1, 0)]
            ng = valid & (j > 0) & (tj != tp)
            fi = m_s[b, j]
            dv = wd_s[fi] * d_s[b, j] + bd_s[fi]
            valid_l.append(jnp.where(valid, 1.0, 0.0))
            ng_l.append(jnp.where(ng, 1.0, 0.0))
            fi_l.append(fi)
            dv_l.append(dv)
            xj_l.append(x_s[b, j])
        vmask = _rows(valid_l)
        ngmask = _rows(ng_l)
        decay = jnp.exp(-jnp.maximum(_rows(dv_l), 0.0))
        xv = _rows(xj_l, 4 * H)

        c_all = ngmask * (s_all / jnp.maximum(cnt_all, 1.0)) \
            + (1.0 - ngmask) * c_all
        s_all = (1.0 - ngmask) * s_all
        cnt_all = (1.0 - ngmask) * cnt_all

        h_rows = jnp.concatenate(
            [h_ref[b, pl.ds(fi_l[b], 1), :] for b in range(B)], axis=0)
        h_dec = decay * h_rows
        h_bf = h_dec.astype(jnp.bfloat16)
        outs = jnp.concatenate(
            [jax.lax.dot_general(
                h_bf[b:b + 1], W1[fi_l[b]], (((1,), (0,)), ((), ())),
                preferred_element_type=jnp.float32) for b in range(B)],
            axis=0)
        w0 = jnp.concatenate(
            [W0[pl.ds(fi_l[b], 1), :] for b in range(B)], axis=0)
        blv = jnp.concatenate(
            [bl[pl.ds(fi_l[b], 1), :] for b in range(B)], axis=0)
        outs = outs + xv * w0 + blv

        gi = jax.nn.sigmoid(outs[:, 0:H])
        gf = jax.nn.sigmoid(outs[:, H:2 * H])
        go = jax.nn.sigmoid(outs[:, 2 * H:3 * H])
        gc = jnp.tanh(outs[:, 3 * H:4 * H])
        cell = gf * c_all + gi * gc
        hnew = go * jnp.tanh(cell)
        hmix = vmask * hnew + (1.0 - vmask) * h_rows
        for b in range(B):
            h_ref[b, pl.ds(fi_l[b], 1), :] = hmix[b:b + 1]
        s_all = s_all + vmask * cell
        cnt_all = cnt_all + vmask
        return c_all, s_all, cnt_all

    init = (jnp.zeros((B, H), jnp.float32),
            jnp.zeros((B, H), jnp.float32),
            jnp.zeros((B, H), jnp.float32))
    c_all, s_all, cnt_all = jax.lax.fori_loop(0, n_steps, step, init)
    c_all = s_all / cnt_all

    for b in range(B):
        feat = jnp.concatenate([c_all[b:b + 1], h_ref[b]], axis=0)
        l0 = jnp.sum(feat * WoT[0])
        l1 = jnp.sum(feat * WoT[1])
        d = (l1 - l0) + (bo_s[1] - bo_s[0])
        p1 = jax.nn.sigmoid(jnp.full((1, H), d))
        out_ref[b:b + 1, 0:1] = (1.0 - p1)[:, 0:1]
        out_ref[b:b + 1, 1:2] = p1[:, 0:1]


def kernel(X, lengths, W_l, b_l, w_d, b_d, W_o, b_o):
    t = X[:, 0, :]
    m = X[:, 1, :].astype(jnp.int32)
    x = X[:, 2, :]
    delt = X[:, 3, :]
    W0 = W_l[:, 0, :]
    W1 = W_l[:, 1:, :].astype(jnp.bfloat16)
    WoT = W_o.reshape(F + 1, H, C).transpose(2, 0, 1)

    smem = pl.BlockSpec(memory_space=pltpu.SMEM)
    vmem = pl.BlockSpec(memory_space=pltpu.VMEM)
    out = pl.pallas_call(
        _body,
        out_shape=jax.ShapeDtypeStruct((B, C), jnp.float32),
        in_specs=[smem, smem, smem, smem, smem, smem, smem,
                  vmem, vmem, vmem, vmem, smem],
        out_specs=pl.BlockSpec(memory_space=pltpu.VMEM),
        scratch_shapes=[pltpu.VMEM((B, F, H), jnp.float32)],
    )(t, m, x, delt, lengths.astype(jnp.int32), w_d, b_d,
      W0, W1, b_l, WoT, b_o)
    return out


# speedup vs baseline: 38.0847x; 1.2004x over previous
import jax
import jax.numpy as jnp
from jax.experimental import pallas as pl
from jax.experimental.pallas import tpu as pltpu

B, T, F, H, C = 4, 256, 64, 128, 2
K = 4


def _rows(scalars, width=H):
    return jnp.concatenate(
        [jnp.full((1, width), s, jnp.float32) for s in scalars], axis=0)


def _body(t_s, m_s, x_s, d_s, len_s, wd_s, bd_s,
          W0, W1, bl, WoT, bo_s, out_ref, h_ref, outs_ref):
    h_ref[...] = jnp.zeros((B, F, H), dtype=jnp.float32)
    n_steps = jnp.maximum(jnp.maximum(len_s[0], len_s[1]),
                          jnp.maximum(len_s[2], len_s[3]))
    n_blocks = (n_steps + (K - 1)) // K

    def matvec_all(j, fi_l):
        dv_l = [wd_s[fi_l[b]] * d_s[b, j] + bd_s[fi_l[b]] for b in range(B)]
        decay = jnp.exp(-jnp.maximum(_rows(dv_l), 0.0))
        h_rows = jnp.concatenate(
            [h_ref[b, pl.ds(fi_l[b], 1), :] for b in range(B)], axis=0)
        h_bf = (decay * h_rows).astype(jnp.bfloat16)
        outs = jnp.concatenate(
            [jax.lax.dot_general(
                h_bf[b:b + 1], W1[fi_l[b]], (((1,), (0,)), ((), ())),
                preferred_element_type=jnp.float32) for b in range(B)],
            axis=0)
        w0 = jnp.concatenate(
            [W0[pl.ds(fi_l[b], 1), :] for b in range(B)], axis=0)
        blv = jnp.concatenate(
            [bl[pl.ds(fi_l[b], 1), :] for b in range(B)], axis=0)
        xv = _rows([x_s[b, j] for b in range(B)], 4 * H)
        return outs + xv * w0 + blv, h_rows

    def block(i, carry):
        c_all, s_all, cnt_all = carry
        j0 = i * K
        fis = [[m_s[b, j0 + k] for b in range(B)] for k in range(K)]

        for k in range(K):
            outs_k, _ = matvec_all(j0 + k, fis[k])
            outs_ref[pl.ds(k * B, B), :] = outs_k

        for k in range(K):
            j = j0 + k
            if k > 0:
                dup = False
                for b in range(B):
                    d_b = fis[k][b] != fis[k][b]
                    for kp in range(k):
                        d_b = d_b | (fis[k][b] == fis[kp][b])
                    dup = dup | d_b

                @pl.when(dup)
                def _():
                    outs_k, _ = matvec_all(j, fis[k])
                    outs_ref[pl.ds(k * B, B), :] = outs_k

            valid_l, ng_l = [], []
            for b in range(B):
                valid = j < len_s[b]
                tj = t_s[b, j]
                tp = t_s[b, jnp.maximum(j - 1, 0)]
                ng = valid & (j > 0) & (tj != tp)
                valid_l.append(jnp.where(valid, 1.0, 0.0))
                ng_l.append(jnp.where(ng, 1.0, 0.0))
            vmask = _rows(valid_l)
            ngmask = _rows(ng_l)

            c_all = ngmask * (s_all / jnp.maximum(cnt_all, 1.0)) \
                + (1.0 - ngmask) * c_all
            s_all = (1.0 - ngmask) * s_all
            cnt_all = (1.0 - ngmask) * cnt_all

            outs = outs_ref[pl.ds(k * B, B), :]
            sg = jax.nn.sigmoid(outs[:, 0:3 * H])
            gi = sg[:, 0:H]
            gf = sg[:, H:2 * H]
            go = sg[:, 2 * H:3 * H]
            gc = jnp.tanh(outs[:, 3 * H:4 * H])
            cell = gf * c_all + gi * gc
            hnew = go * jnp.tanh(cell)
            for b in range(B):
                h_old = h_ref[b, pl.ds(fis[k][b], 1), :]
                h_ref[b, pl.ds(fis[k][b], 1), :] = \
                    vmask[b:b + 1] * hnew[b:b + 1] \
                    + (1.0 - vmask[b:b + 1]) * h_old
            s_all = s_all + vmask * cell
            cnt_all = cnt_all + vmask
        return c_all, s_all, cnt_all

    init = (jnp.zeros((B, H), jnp.float32),
            jnp.zeros((B, H), jnp.float32),
            jnp.zeros((B, H), jnp.float32))
    c_all, s_all, cnt_all = jax.lax.fori_loop(0, n_blocks, block, init)
    c_all = s_all / cnt_all

    for b in range(B):
        feat = jnp.concatenate([c_all[b:b + 1], h_ref[b]], axis=0)
        l0 = jnp.sum(feat * WoT[0])
        l1 = jnp.sum(feat * WoT[1])
        d = (l1 - l0) + (bo_s[1] - bo_s[0])
        p1 = jax.nn.sigmoid(jnp.full((1, H), d))
        out_ref[b:b + 1, 0:1] = (1.0 - p1)[:, 0:1]
        out_ref[b:b + 1, 1:2] = p1[:, 0:1]


def kernel(X, lengths, W_l, b_l, w_d, b_d, W_o, b_o):
    t = X[:, 0, :]
    m = X[:, 1, :].astype(jnp.int32)
    x = X[:, 2, :]
    delt = X[:, 3, :]
    W0 = W_l[:, 0, :]
    W1 = W_l[:, 1:, :].astype(jnp.bfloat16)
    WoT = W_o.reshape(F + 1, H, C).transpose(2, 0, 1)

    smem = pl.BlockSpec(memory_space=pltpu.SMEM)
    vmem = pl.BlockSpec(memory_space=pltpu.VMEM)
    out = pl.pallas_call(
        _body,
        out_shape=jax.ShapeDtypeStruct((B, C), jnp.float32),
        in_specs=[smem, smem, smem, smem, smem, smem, smem,
                  vmem, vmem, vmem, vmem, smem],
        out_specs=pl.BlockSpec(memory_space=pltpu.VMEM),
        scratch_shapes=[pltpu.VMEM((B, F, H), jnp.float32),
                        pltpu.VMEM((K * B, 4 * H), jnp.float32)],
    )(t, m, x, delt, lengths.astype(jnp.int32), w_d, b_d,
      W0, W1, b_l, WoT, b_o)
    return out
